# baseline (speedup 1.0000x reference)
import functools

import jax
import jax.numpy as jnp
from jax import lax
from jax.experimental import pallas as pl
from jax.experimental.pallas import tpu as pltpu
from jax.experimental.pallas import tpu_sc as plsc

_D, _H, _W = 32, 32, 32
_C = 128
_CW = _C // 2
_HW = _H * _W
_NC, _NS = 2, 16
_NW = _NC * _NS
_CH = 64
_L = 16
_G = 8

_CORNERS = (
    (0, 0, 0),
    (1, 0, 1),
    (0, _CW, 2),
    (1, _CW, 3),
    (2, 0, 4),
    (3, 0, 5),
    (2, _CW, 6),
    (3, _CW, 7),
)

_BCAST_DNUMS = lax.GatherDimensionNumbers(
    offset_dims=(), collapsed_slice_dims=(0,), start_index_map=(0,))


def _bcast_lane(v, i):
    idx = jnp.full((_L, 1), i, jnp.int32)
    return lax.gather(v, idx, _BCAST_DNUMS, slice_sizes=(1,),
                      mode=lax.GatherScatterMode.PROMISE_IN_BOUNDS)


def _axis_prep(coord, dimlen):
    g = 2.0 * coord / (dimlen - 1.0) - 1.0
    v = jnp.clip((g + 1.0) * 0.5 * (dimlen - 1.0), 0.0, dimlen - 1.0)
    i0 = v.astype(jnp.int32)
    w = v - i0.astype(jnp.float32)
    i1 = jnp.minimum(i0 + 1, dimlen - 1)
    return i0, i1, w


def _bf16_bits(f):
    u = lax.bitcast_convert_type(f, jnp.uint32)
    return (u + jnp.uint32(0x7FFF) + ((u >> 16) & jnp.uint32(1))) >> 16


def _pack_words(plane_t):
    lo = _bf16_bits(plane_t[:, :_CW])
    hi = _bf16_bits(plane_t[:, _CW:])
    return lax.bitcast_convert_type(lo | (hi << 16), jnp.int32)


_ZB = 2


def _pack_kernel(vol_ref, nxt_ref, out_ref):
    own = []
    for z in range(_ZB):
        plane = jnp.transpose(
            vol_ref[0, :, z, :, :], (1, 2, 0)).reshape(_HW, _C)
        own.append(_pack_words(plane))
    nrow = _pack_words(
        jnp.transpose(nxt_ref[0, :, 0, 0, :], (1, 0)))
    for z in range(_ZB):
        nxt = own[z + 1][:_W] if z + 1 < _ZB else nrow
        neigh = jnp.concatenate([own[z][_W:], nxt], axis=0)
        out_ref[0, z] = jnp.concatenate([own[z], neigh], axis=1)


def _build_table(encoder_outputs):
    b = encoder_outputs.shape[0]
    nzb = _D // _ZB
    grid = (b * nzb,)
    return pl.pallas_call(
        _pack_kernel,
        grid=grid,
        in_specs=[
            pl.BlockSpec((1, _C, _ZB, _H, _W),
                         lambda g: (g // nzb, 0, g % nzb, 0, 0)),
            pl.BlockSpec((1, _C, _ZB, _H, _W),
                         lambda g: (g // nzb, 0,
                                    jnp.minimum(g % nzb + 1, nzb - 1), 0, 0)),
        ],
        out_specs=pl.BlockSpec((1, _ZB, _HW, _C), lambda g: (g, 0, 0, 0)),
        out_shape=jax.ShapeDtypeStruct((b * nzb, _ZB, _HW, _C), jnp.int32),
    )(encoder_outputs, encoder_outputs).reshape(b * _D * _HW, _C)


def _make_sc_kernel(n_chunks):
    n, t_chunks = n_chunks
    mesh = plsc.VectorSubcoreMesh(core_axis_name="c", subcore_axis_name="s")

    per_w = -(-t_chunks // _NW)
    n_per_w = _CH * per_w
    scratch = (
        [pltpu.VMEM((n_per_w,), jnp.float32) for _ in range(3)]
        + [pltpu.VMEM((n_per_w,), jnp.int32)]
        + [pltpu.VMEM((_CH,), jnp.int32) for _ in range(8)]
        + [pltpu.VMEM((_CH,), jnp.float32) for _ in range(16)]
        + [pltpu.VMEM((_CH, _C), jnp.int32) for _ in range(8)]
        + [pltpu.VMEM((_CH, _C), jnp.float32) for _ in range(2)]
        + [pltpu.SemaphoreType.DMA, pltpu.SemaphoreType.DMA]
        + [pltpu.SemaphoreType.DMA, pltpu.SemaphoreType.DMA]
    )

    @functools.partial(
        pl.kernel,
        mesh=mesh,
        out_type=jax.ShapeDtypeStruct((n, _C), jnp.float32),
        scratch_types=scratch,
    )
    def sc_kernel(xs_h, ys_h, zs_h, bs_h, table_h, out_h, *refs):
        xv, yv, zv = refs[0:3]
        bv = refs[3]
        idx = (refs[4:8], refs[8:12])
        wgt = (refs[12:20], refs[20:28])
        rows = (refs[28:32], refs[32:36])
        ov = refs[36:38]
        sem = refs[38:40]
        osem = refs[40:42]

        wid = lax.axis_index("s") * _NC + lax.axis_index("c")
        wbase = wid * n_per_w
        cnt = jnp.minimum(per_w, t_chunks - wid * per_w)

        wb8 = pl.multiple_of(wbase, 8)
        pltpu.sync_copy(xs_h.at[pl.ds(wb8, n_per_w)], xv)
        pltpu.sync_copy(ys_h.at[pl.ds(wb8, n_per_w)], yv)
        pltpu.sync_copy(zs_h.at[pl.ds(wb8, n_per_w)], zv)
        pltpu.sync_copy(bs_h.at[pl.ds(wb8, n_per_w)], bv)

        def load_and_fire(gi, s):
            cbase = pl.multiple_of(gi * _CH, 8)
            for i in range(_CH // _L):
                sl = pl.ds(cbase + i * _L, _L)
                sw = pl.ds(i * _L, _L)
                x0, x1, wx = _axis_prep(xv[sl], _W)
                y0, _, wy = _axis_prep(yv[sl], _H)
                z0, z1, wz = _axis_prep(zv[sl], _D)
                bb = bv[sl]
                r0 = ((bb * _D + z0) * _H + y0) * _W
                r1 = ((bb * _D + z1) * _H + y0) * _W
                idx[s][0][sw] = r0 + x0
                idx[s][1][sw] = r0 + x1
                idx[s][2][sw] = r1 + x0
                idx[s][3][sw] = r1 + x1
                ux = 1.0 - wx
                uy = 1.0 - wy
                uz = 1.0 - wz
                wgt[s][0][sw] = uz * uy * ux
                wgt[s][1][sw] = uz * uy * wx
                wgt[s][2][sw] = uz * wy * ux
                wgt[s][3][sw] = uz * wy * wx
                wgt[s][4][sw] = wz * uy * ux
                wgt[s][5][sw] = wz * uy * wx
                wgt[s][6][sw] = wz * wy * ux
                wgt[s][7][sw] = wz * wy * wx

            for k in range(4):
                pltpu.async_copy(table_h.at[idx[s][k]], rows[s][k], sem[s])

        def drain(s):
            for k in range(4):
                pltpu.make_async_copy(
                    table_h.at[idx[s][k]], rows[s][k], sem[s]).wait()

        def accumulate(gi, s):
            base = pl.multiple_of(wbase + gi * _CH, 8)

            def group_body(t, c2):
                gb = pl.multiple_of(t * _G, _G)
                wvec = [wgt[s][k][pl.ds(gb, _L)] for k in range(8)]
                for i in range(_G):
                    nn = gb + i
                    w8 = [_bcast_lane(wvec[k], i) for k in range(8)]
                    for j in range(_CW // _L):
                        pe = []
                        po = []
                        for (buf, woff, k) in _CORNERS:
                            wv = rows[s][buf][nn, pl.ds(woff + j * _L, _L)]
                            ev = lax.bitcast_convert_type(
                                lax.shift_left(wv, 16), jnp.float32)
                            od = lax.bitcast_convert_type(wv, jnp.float32)
                            pe.append(ev * w8[k])
                            po.append(od * w8[k])
                        acc_e = ((pe[0] + pe[1]) + (pe[2] + pe[3])) + (
                            (pe[4] + pe[5]) + (pe[6] + pe[7]))
                        acc_o = ((po[0] + po[1]) + (po[2] + po[3])) + (
                            (po[4] + po[5]) + (po[6] + po[7]))
                        ov[s][nn, pl.ds(j * _L, _L)] = acc_e
                        ov[s][nn, pl.ds(_CW + j * _L, _L)] = acc_o
                return c2

            lax.fori_loop(0, _CH // _G, group_body, 0, unroll=False)
            pltpu.async_copy(ov[s], out_h.at[pl.ds(base, _CH)], osem[s])

        def drain_out(gi, s):
            base = pl.multiple_of(wbase + gi * _CH, 8)
            pltpu.make_async_copy(
                ov[s], out_h.at[pl.ds(base, _CH)], osem[s]).wait()

        load_and_fire(0, 0)

        def outer(gp, carry):
            for b in range(2):
                g = 2 * gp + b
                nxt = g + 1

                @pl.when(nxt < cnt)
                def _():
                    load_and_fire(nxt, (b + 1) % 2)

                drain(b)

                @pl.when(gp >= 1)
                def _():
                    drain_out(g - 2, b)

                accumulate(g, b)
            return carry

        lax.fori_loop(0, cnt // 2, outer, 0, unroll=False)

        @pl.when(cnt % 2 == 1)
        def _():
            g = cnt - 1
            drain(0)

            @pl.when(cnt >= 3)
            def _():
                drain_out(g - 2, 0)

            accumulate(g, 0)

        @pl.when(cnt % 2 == 0)
        def _():
            drain_out(cnt - 2, 0)
            drain_out(cnt - 1, 1)

        @pl.when(cnt % 2 == 1)
        def _():
            drain_out(cnt - 2, 1)
            drain_out(cnt - 1, 0)

    return sc_kernel


def kernel(encoder_outputs, graph_coords, batch):
    n = graph_coords.shape[0]
    assert n % _CH == 0
    t_chunks = n // _CH
    per_w = -(-t_chunks // _NW)
    n_coord = _NW * per_w * _CH
    pad = n_coord - n

    table_w = _build_table(encoder_outputs)
    xs = jnp.pad(graph_coords[:, 0], (0, pad))
    ys = jnp.pad(graph_coords[:, 1], (0, pad))
    zs = jnp.pad(graph_coords[:, 2], (0, pad))
    bs = jnp.pad(batch, (0, pad))

    return _make_sc_kernel((n, t_chunks))(xs, ys, zs, bs, table_w)


# speedup vs baseline: 1.7217x; 1.7217x over previous
import functools

import jax
import jax.numpy as jnp
from jax import lax
from jax.experimental import pallas as pl
from jax.experimental.pallas import tpu as pltpu
from jax.experimental.pallas import tpu_sc as plsc

_D, _H, _W = 32, 32, 32
_C = 128
_CW = _C // 2
_HW = _H * _W
_NC, _NS = 2, 16
_NW = _NC * _NS
_CH = 64
_L = 16
_G = 8

_CORNERS = (
    (0, 0, 0),
    (1, 0, 1),
    (0, _CW, 2),
    (1, _CW, 3),
    (2, 0, 4),
    (3, 0, 5),
    (2, _CW, 6),
    (3, _CW, 7),
)

_BCAST_DNUMS = lax.GatherDimensionNumbers(
    offset_dims=(), collapsed_slice_dims=(0,), start_index_map=(0,))


def _bcast_lane(v, i):
    idx = jnp.full((_L, 1), i, jnp.int32)
    return lax.gather(v, idx, _BCAST_DNUMS, slice_sizes=(1,),
                      mode=lax.GatherScatterMode.PROMISE_IN_BOUNDS)


def _axis_prep(coord, dimlen):
    g = 2.0 * coord / (dimlen - 1.0) - 1.0
    v = jnp.clip((g + 1.0) * 0.5 * (dimlen - 1.0), 0.0, dimlen - 1.0)
    i0 = v.astype(jnp.int32)
    w = v - i0.astype(jnp.float32)
    i1 = jnp.minimum(i0 + 1, dimlen - 1)
    return i0, i1, w


def _bf16_bits(f):
    u = lax.bitcast_convert_type(f, jnp.uint32)
    return (u + jnp.uint32(0x7FFF) + ((u >> 16) & jnp.uint32(1))) >> 16


def _pack_words(plane_t):
    lo = _bf16_bits(plane_t[:, :_CW])
    hi = _bf16_bits(plane_t[:, _CW:])
    return lax.bitcast_convert_type(lo | (hi << 16), jnp.int32)


_ZB = 8


def _pack_kernel(vol_ref, nxt_ref, out_ref):
    own = []
    for z in range(_ZB):
        plane = jnp.transpose(vol_ref[0, :, z, :])
        own.append(_pack_words(plane))
    nrow = _pack_words(jnp.transpose(nxt_ref[0, :, 0, :_W]))
    for z in range(_ZB):
        nxt = own[z + 1][:_W] if z + 1 < _ZB else nrow
        neigh = jnp.concatenate([own[z][_W:], nxt], axis=0)
        out_ref[0, z] = jnp.concatenate([own[z], neigh], axis=1)


def _build_table(encoder_outputs):
    b = encoder_outputs.shape[0]
    vol = encoder_outputs.reshape(b, _C, _D, _HW)
    nzb = _D // _ZB
    grid = (b * nzb,)
    return pl.pallas_call(
        _pack_kernel,
        grid=grid,
        in_specs=[
            pl.BlockSpec((1, _C, _ZB, _HW),
                         lambda g: (g // nzb, 0, g % nzb, 0)),
            pl.BlockSpec((1, _C, _ZB, _HW),
                         lambda g: (g // nzb, 0,
                                    jnp.minimum(g % nzb + 1, nzb - 1), 0)),
        ],
        out_specs=pl.BlockSpec((1, _ZB, _HW, _C), lambda g: (g, 0, 0, 0)),
        out_shape=jax.ShapeDtypeStruct((b * nzb, _ZB, _HW, _C), jnp.int32),
    )(vol, vol).reshape(b * _D * _HW, _C)


def _make_sc_kernel(n_chunks):
    n, t_chunks = n_chunks
    mesh = plsc.VectorSubcoreMesh(core_axis_name="c", subcore_axis_name="s")

    per_w = -(-t_chunks // _NW)
    n_per_w = _CH * per_w
    scratch = (
        [pltpu.VMEM((n_per_w,), jnp.float32) for _ in range(3)]
        + [pltpu.VMEM((n_per_w,), jnp.int32)]
        + [pltpu.VMEM((_CH,), jnp.int32) for _ in range(8)]
        + [pltpu.VMEM((_CH,), jnp.float32) for _ in range(16)]
        + [pltpu.VMEM((_CH, _C), jnp.int32) for _ in range(8)]
        + [pltpu.VMEM((_CH, _C), jnp.float32) for _ in range(2)]
        + [pltpu.SemaphoreType.DMA, pltpu.SemaphoreType.DMA]
        + [pltpu.SemaphoreType.DMA, pltpu.SemaphoreType.DMA]
    )

    @functools.partial(
        pl.kernel,
        mesh=mesh,
        out_type=jax.ShapeDtypeStruct((n, _C), jnp.float32),
        scratch_types=scratch,
    )
    def sc_kernel(xs_h, ys_h, zs_h, bs_h, table_h, out_h, *refs):
        xv, yv, zv = refs[0:3]
        bv = refs[3]
        idx = (refs[4:8], refs[8:12])
        wgt = (refs[12:20], refs[20:28])
        rows = (refs[28:32], refs[32:36])
        ov = refs[36:38]
        sem = refs[38:40]
        osem = refs[40:42]

        wid = lax.axis_index("s") * _NC + lax.axis_index("c")
        wbase = wid * n_per_w
        cnt = jnp.minimum(per_w, t_chunks - wid * per_w)

        wb8 = pl.multiple_of(wbase, 8)
        pltpu.sync_copy(xs_h.at[pl.ds(wb8, n_per_w)], xv)
        pltpu.sync_copy(ys_h.at[pl.ds(wb8, n_per_w)], yv)
        pltpu.sync_copy(zs_h.at[pl.ds(wb8, n_per_w)], zv)
        pltpu.sync_copy(bs_h.at[pl.ds(wb8, n_per_w)], bv)

        def load_and_fire(gi, s):
            cbase = pl.multiple_of(gi * _CH, 8)
            for i in range(_CH // _L):
                sl = pl.ds(cbase + i * _L, _L)
                sw = pl.ds(i * _L, _L)
                x0, x1, wx = _axis_prep(xv[sl], _W)
                y0, _, wy = _axis_prep(yv[sl], _H)
                z0, z1, wz = _axis_prep(zv[sl], _D)
                bb = bv[sl]
                r0 = ((bb * _D + z0) * _H + y0) * _W
                r1 = ((bb * _D + z1) * _H + y0) * _W
                idx[s][0][sw] = r0 + x0
                idx[s][1][sw] = r0 + x1
                idx[s][2][sw] = r1 + x0
                idx[s][3][sw] = r1 + x1
                ux = 1.0 - wx
                uy = 1.0 - wy
                uz = 1.0 - wz
                wgt[s][0][sw] = uz * uy * ux
                wgt[s][1][sw] = uz * uy * wx
                wgt[s][2][sw] = uz * wy * ux
                wgt[s][3][sw] = uz * wy * wx
                wgt[s][4][sw] = wz * uy * ux
                wgt[s][5][sw] = wz * uy * wx
                wgt[s][6][sw] = wz * wy * ux
                wgt[s][7][sw] = wz * wy * wx

            for k in range(4):
                pltpu.async_copy(table_h.at[idx[s][k]], rows[s][k], sem[s])

        def drain(s):
            for k in range(4):
                pltpu.make_async_copy(
                    table_h.at[idx[s][k]], rows[s][k], sem[s]).wait()

        def accumulate(gi, s):
            base = pl.multiple_of(wbase + gi * _CH, 8)

            def group_body(t, c2):
                gb = pl.multiple_of(t * _G, _G)
                wvec = [wgt[s][k][pl.ds(gb, _L)] for k in range(8)]
                for i in range(_G):
                    nn = gb + i
                    w8 = [_bcast_lane(wvec[k], i) for k in range(8)]
                    for j in range(_CW // _L):
                        pe = []
                        po = []
                        for (buf, woff, k) in _CORNERS:
                            wv = rows[s][buf][nn, pl.ds(woff + j * _L, _L)]
                            ev = lax.bitcast_convert_type(
                                lax.shift_left(wv, 16), jnp.float32)
                            od = lax.bitcast_convert_type(wv, jnp.float32)
                            pe.append(ev * w8[k])
                            po.append(od * w8[k])
                        acc_e = ((pe[0] + pe[1]) + (pe[2] + pe[3])) + (
                            (pe[4] + pe[5]) + (pe[6] + pe[7]))
                        acc_o = ((po[0] + po[1]) + (po[2] + po[3])) + (
                            (po[4] + po[5]) + (po[6] + po[7]))
                        ov[s][nn, pl.ds(j * _L, _L)] = acc_e
                        ov[s][nn, pl.ds(_CW + j * _L, _L)] = acc_o
                return c2

            lax.fori_loop(0, _CH // _G, group_body, 0, unroll=False)
            pltpu.async_copy(ov[s], out_h.at[pl.ds(base, _CH)], osem[s])

        def drain_out(gi, s):
            base = pl.multiple_of(wbase + gi * _CH, 8)
            pltpu.make_async_copy(
                ov[s], out_h.at[pl.ds(base, _CH)], osem[s]).wait()

        load_and_fire(0, 0)

        def outer(gp, carry):
            for b in range(2):
                g = 2 * gp + b
                nxt = g + 1

                @pl.when(nxt < cnt)
                def _():
                    load_and_fire(nxt, (b + 1) % 2)

                drain(b)

                @pl.when(gp >= 1)
                def _():
                    drain_out(g - 2, b)

                accumulate(g, b)
            return carry

        lax.fori_loop(0, cnt // 2, outer, 0, unroll=False)

        @pl.when(cnt % 2 == 1)
        def _():
            g = cnt - 1
            drain(0)

            @pl.when(cnt >= 3)
            def _():
                drain_out(g - 2, 0)

            accumulate(g, 0)

        @pl.when(cnt % 2 == 0)
        def _():
            drain_out(cnt - 2, 0)
            drain_out(cnt - 1, 1)

        @pl.when(cnt % 2 == 1)
        def _():
            drain_out(cnt - 2, 1)
            drain_out(cnt - 1, 0)

    return sc_kernel


def kernel(encoder_outputs, graph_coords, batch):
    n = graph_coords.shape[0]
    assert n % _CH == 0
    t_chunks = n // _CH
    per_w = -(-t_chunks // _NW)
    n_coord = _NW * per_w * _CH
    pad = n_coord - n

    table_w = _build_table(encoder_outputs)
    xs = jnp.pad(graph_coords[:, 0], (0, pad))
    ys = jnp.pad(graph_coords[:, 1], (0, pad))
    zs = jnp.pad(graph_coords[:, 2], (0, pad))
    bs = jnp.pad(batch, (0, pad))

    return _make_sc_kernel((n, t_chunks))(xs, ys, zs, bs, table_w)
